# baseline (speedup 1.0000x reference)
import functools

import jax
import jax.numpy as jnp
from jax import lax
from jax.experimental import pallas as pl
from jax.experimental.pallas import tpu as pltpu
from jax.experimental.pallas import tpu_sc as plsc

HIDDEN = 64
B, L = 1024, 200
ROWS = B * L
NW = 32
RPW = ROWS // NW
CH = 64
NCH = RPW // CH
NB = 5
NBLK = NCH // NB
OCH = CH // 2
V2 = 500000


def _i16():
    return jnp.arange(16, dtype=jnp.int32)


def _splat(v):
    return jnp.full((16,), v, dtype=jnp.int32)


@functools.cache
def _make_fused():
    mesh = plsc.VectorSubcoreMesh(core_axis_name="c", subcore_axis_name="s")

    @functools.partial(
        pl.kernel,
        mesh=mesh,
        out_type=jax.ShapeDtypeStruct((ROWS // 2, 128), jnp.float32),
        scratch_types=[
            pltpu.VMEM((NCH, CH), jnp.int32),
            pltpu.VMEM((NCH, CH), jnp.int32),
            pltpu.VMEM((2 * NB, CH, 128), jnp.float32),
            pltpu.VMEM((2, OCH, 128), jnp.float32),
            pltpu.VMEM((L // 2, 128), jnp.float32),
            pltpu.VMEM((1024,), jnp.float32),
            pltpu.VMEM((128,), jnp.float32),
            pltpu.SemaphoreType.DMA((2 * NB,)),
            pltpu.SemaphoreType.DMA((2,)),
        ],
        compiler_params=pltpu.CompilerParams(
            use_tc_tiling_on_sc=False, needs_layout_passes=False),
    )
    def fused_k(idr_hbm, idh_hbm, tab_hbm, pos_hbm, gb_hbm, out_hbm,
                idr_v, idh_v, rows_v, wb_v, pos_v, txp_v, gb_v,
                gsems, wsems):
        wid = lax.axis_index("s") * 2 + lax.axis_index("c")
        pltpu.sync_copy(idr_hbm.at[wid], idr_v)
        pltpu.sync_copy(idh_hbm.at[wid], idh_v)
        pltpu.sync_copy(pos_hbm, pos_v)
        pltpu.sync_copy(gb_hbm, gb_v)
        rbase = wid * RPW
        obase = wid * (RPW // 2)

        def fire_block(k, s):
            for b in range(NB):
                slot = s * NB + b
                pltpu.async_copy(tab_hbm.at[idh_v.at[k * NB + b]],
                                 rows_v.at[slot], gsems.at[slot])

        def wait_gather(slot):
            pltpu.make_async_copy(tab_hbm.at[idh_v.at[0]],
                                  rows_v.at[slot], gsems.at[slot]).wait()

        def wait_wb(ws):
            pltpu.make_async_copy(wb_v.at[ws], out_hbm.at[pl.ds(0, OCH)],
                                  wsems.at[ws]).wait()

        def newton_rsqrt(v):
            u = plsc.bitcast(v, jnp.int32)
            u = 0x5F3759DF - lax.shift_right_logical(u, 1)
            y = plsc.bitcast(u, jnp.float32)
            for _ in range(3):
                y = y * (1.5 - 0.5 * v * y * y)
            return y

        def compute_chunk(jg, slot, ws):
            rowbase = rbase + jg * CH
            jg16 = _splat(jg)
            slot16 = _splat(slot)
            ws16 = _splat(ws)

            def group(g, gcarry):
                rb = _i16() + g * 16
                grow = rowbase + rb
                gl = lax.rem(grow, L)
                lr = lax.shift_right_logical(gl, 1)
                lc = (gl & 1) * 64
                idsr = plsc.load_gather(idr_v, [jg16, rb])
                par = (idsr & 1) * 64

                def pass1(h, carry):
                    acc, acc2 = carry
                    xh = plsc.load_gather(rows_v, [slot16, rb, par + h])
                    ph = plsc.load_gather(pos_v, [lr, lc + h])
                    yh = xh + ph
                    txp_v[pl.ds(h * 16, 16)] = yh
                    return acc + yh, acc2 + yh * yh

                z = jnp.zeros((16,), jnp.float32)
                acc, acc2 = lax.fori_loop(0, HIDDEN, pass1, (z, z), unroll=4)
                mean = acc * (1.0 / HIDDEN)
                var = acc2 * (1.0 / HIDDEN) - mean * mean + 1e-5
                rinv = newton_rsqrt(var)
                m2 = mean * rinv
                wr = lax.shift_right_logical(rb, 1)
                wc = (rb & 1) * 64

                def pass2(h, carry):
                    yh = txp_v[pl.ds(h * 16, 16)]
                    oh = yh * rinv - m2
                    gh = plsc.load_gather(gb_v, [_splat(h)])
                    bh = plsc.load_gather(gb_v, [_splat(64 + h)])
                    oh = oh * gh + bh
                    plsc.store_scatter(wb_v, [ws16, wr, wc + h], oh)
                    return carry

                lax.fori_loop(0, HIDDEN, pass2, 0, unroll=4)
                return gcarry

            lax.fori_loop(0, CH // 16, group, 0)

        def process_block(k, s, first):
            for b in range(NB):
                slot = s * NB + b
                ws = b % 2
                wait_gather(slot)
                if not (first and b < 2):
                    wait_wb(ws)
                jg = k * NB + b
                compute_chunk(jg, slot, ws)
                pltpu.async_copy(wb_v.at[ws],
                                 out_hbm.at[pl.ds(obase + jg * OCH, OCH)],
                                 wsems.at[ws])

        fire_block(0, 0)
        fire_block(1, 1)
        process_block(0, 0, True)

        def body(i, carry):
            k = 2 * i + 1
            fire_block(k + 1, 0)
            process_block(k, 1, False)
            fire_block(k + 2, 1)
            process_block(k + 1, 0, False)
            return carry

        lax.fori_loop(0, (NBLK - 2) // 2, body, 0)
        process_block(NBLK - 1, 1, False)
        wait_wb(0)
        wait_wb(1)

    return fused_k


def kernel(input_ids, word_table, pos_table, ln_gamma, ln_beta):
    ids = input_ids.astype(jnp.int32)
    idr = ids.reshape(NW, NCH, CH)
    idh = lax.shift_right_logical(ids, 1).reshape(NW, NCH, CH)
    tab2 = word_table.reshape(V2, 128)
    pos2 = pos_table[:L].reshape(L // 2, 128)
    gb = jnp.concatenate([ln_gamma, ln_beta])
    out2 = _make_fused()(idr, idh, tab2, pos2, gb)
    return out2.reshape(B, L, HIDDEN)


# speedup vs baseline: 1.3289x; 1.3289x over previous
import functools

import jax
import jax.numpy as jnp
from jax import lax
from jax.experimental import pallas as pl
from jax.experimental.pallas import tpu as pltpu
from jax.experimental.pallas import tpu_sc as plsc

HIDDEN = 64
B, L = 1024, 200
ROWS = B * L
NW = 32
RPW = ROWS // NW
CH = 64
NCH = RPW // CH
NB = 5
NBLK = NCH // NB
OCH = CH // 2
BB = 32
V2 = 500000


def _i16():
    return jnp.arange(16, dtype=jnp.int32)


def _splat(v):
    return jnp.full((16,), v, dtype=jnp.int32)


@functools.cache
def _make_sc_gather():
    mesh = plsc.VectorSubcoreMesh(core_axis_name="c", subcore_axis_name="s")

    @functools.partial(
        pl.kernel,
        mesh=mesh,
        out_type=jax.ShapeDtypeStruct((ROWS // 2, 128), jnp.float32),
        scratch_types=[
            pltpu.VMEM((NCH, CH), jnp.int32),
            pltpu.VMEM((NCH, CH), jnp.int32),
            pltpu.VMEM((2 * NB, CH, 128), jnp.float32),
            pltpu.VMEM((2, OCH, 128), jnp.float32),
            pltpu.SemaphoreType.DMA((2 * NB,)),
            pltpu.SemaphoreType.DMA((2,)),
        ],
        compiler_params=pltpu.CompilerParams(needs_layout_passes=False),
    )
    def gather_k(idr_hbm, idh_hbm, tab_hbm, out_hbm,
                 idr_v, idh_v, rows_v, wb_v, gsems, wsems):
        wid = lax.axis_index("s") * 2 + lax.axis_index("c")
        pltpu.sync_copy(idr_hbm.at[wid], idr_v)
        pltpu.sync_copy(idh_hbm.at[wid], idh_v)
        obase = wid * (RPW // 2)

        def fire_block(k, s):
            for b in range(NB):
                slot = s * NB + b
                pltpu.async_copy(tab_hbm.at[idh_v.at[k * NB + b]],
                                 rows_v.at[slot], gsems.at[slot])

        def wait_gather(slot):
            pltpu.make_async_copy(tab_hbm.at[idh_v.at[0]],
                                  rows_v.at[slot], gsems.at[slot]).wait()

        def wait_wb(ws):
            pltpu.make_async_copy(wb_v.at[ws], out_hbm.at[pl.ds(0, OCH)],
                                  wsems.at[ws]).wait()

        def shuffle_chunk(jg, slot, ws):
            jg16 = _splat(jg)
            slot16 = _splat(slot)
            ws16 = _splat(ws)

            def group(g, carry):
                rb = _i16() + g * 16
                idsr = plsc.load_gather(idr_v, [jg16, rb])
                par = (idsr & 1) * 64
                wr = lax.shift_right_logical(rb, 1)
                wc = (rb & 1) * 64
                for h in range(HIDDEN):
                    xh = plsc.load_gather(rows_v, [slot16, rb, par + h])
                    plsc.store_scatter(wb_v, [ws16, wr, wc + h], xh)
                return carry

            lax.fori_loop(0, CH // 16, group, 0)

        def process_block(k, s, first):
            for b in range(NB):
                slot = s * NB + b
                ws = b % 2
                wait_gather(slot)
                if not (first and b < 2):
                    wait_wb(ws)
                jg = k * NB + b
                shuffle_chunk(jg, slot, ws)
                pltpu.async_copy(wb_v.at[ws],
                                 out_hbm.at[pl.ds(obase + jg * OCH, OCH)],
                                 wsems.at[ws])

        fire_block(0, 0)
        fire_block(1, 1)
        process_block(0, 0, True)

        def body(i, carry):
            k = 2 * i + 1
            fire_block(k + 1, 0)
            process_block(k, 1, False)
            fire_block(k + 2, 1)
            process_block(k + 1, 0, False)
            return carry

        lax.fori_loop(0, (NBLK - 2) // 2, body, 0)
        process_block(NBLK - 1, 1, False)
        wait_wb(0)
        wait_wb(1)

    return gather_k


def _ln_body(x_ref, pos_ref, g_ref, b_ref, o_ref):
    x = x_ref[...] + pos_ref[...][None, :, :]

    def norm(v):
        mean = jnp.mean(v, axis=-1, keepdims=True)
        var = jnp.mean(jnp.square(v - mean), axis=-1, keepdims=True)
        return (v - mean) * lax.rsqrt(var + 1e-5)

    y = jnp.concatenate([norm(x[..., :HIDDEN]), norm(x[..., HIDDEN:])], axis=-1)
    o_ref[...] = y * g_ref[...][None, :, :] + b_ref[...][None, :, :]


def _tc_ln(x3, pos, gamma, beta):
    return pl.pallas_call(
        _ln_body,
        grid=(B // BB,),
        in_specs=[
            pl.BlockSpec((BB, L // 2, 128), lambda i: (i, 0, 0)),
            pl.BlockSpec((L // 2, 128), lambda i: (0, 0)),
            pl.BlockSpec((1, 128), lambda i: (0, 0)),
            pl.BlockSpec((1, 128), lambda i: (0, 0)),
        ],
        out_specs=pl.BlockSpec((BB, L // 2, 128), lambda i: (i, 0, 0)),
        out_shape=jax.ShapeDtypeStruct((B, L // 2, 128), jnp.float32),
    )(x3, pos, gamma, beta)


def kernel(input_ids, word_table, pos_table, ln_gamma, ln_beta):
    ids = input_ids.astype(jnp.int32)
    idr = ids.reshape(NW, NCH, CH)
    idh = lax.shift_right_logical(ids, 1).reshape(NW, NCH, CH)
    tab2 = word_table.reshape(V2, 128)
    pos2 = pos_table[:L].reshape(L // 2, 128)
    g2 = jnp.concatenate([ln_gamma, ln_gamma]).reshape(1, 128)
    b2 = jnp.concatenate([ln_beta, ln_beta]).reshape(1, 128)
    packed = _make_sc_gather()(idr, idh, tab2)
    x3 = packed.reshape(B, L // 2, 128)
    y = _tc_ln(x3, pos2, g2, b2)
    return y.reshape(B, L, HIDDEN)


# speedup vs baseline: 2.0436x; 1.5379x over previous
import functools

import jax
import jax.numpy as jnp
from jax import lax
from jax.experimental import pallas as pl
from jax.experimental.pallas import tpu as pltpu
from jax.experimental.pallas import tpu_sc as plsc

HIDDEN = 64
B, L = 1024, 200
ROWS = B * L
NW = 32
RPW = ROWS // NW
CH = 128
NCH = RPW // CH
NB = 5
NBLK = NCH // NB
BB = 32


@functools.cache
def _make_sc_gather():
    mesh = plsc.VectorSubcoreMesh(core_axis_name="c", subcore_axis_name="s")

    @functools.partial(
        pl.kernel,
        mesh=mesh,
        out_type=jax.ShapeDtypeStruct((ROWS, HIDDEN), jnp.float32),
        scratch_types=[
            pltpu.VMEM((NCH, CH), jnp.int32),
            pltpu.VMEM((2, NB, CH, HIDDEN), jnp.float32),
            pltpu.SemaphoreType.DMA((2, NB)),
            pltpu.SemaphoreType.DMA((2, NB)),
        ],
        compiler_params=pltpu.CompilerParams(use_tc_tiling_on_sc=False),
    )
    def gather_k(ids_hbm, table_hbm, out_hbm, idx_v, rows_v, gsems, wsems):
        wid = lax.axis_index("s") * 2 + lax.axis_index("c")
        pltpu.sync_copy(ids_hbm.at[wid], idx_v)
        base = wid * RPW

        def fire(j, s, b):
            return pltpu.async_copy(
                table_hbm.at[idx_v.at[j]], rows_v.at[s, b], gsems.at[s, b])

        def write(j, s, b):
            pltpu.async_copy(
                rows_v.at[s, b], out_hbm.at[pl.ds(base + j * CH, CH)],
                wsems.at[s, b])

        def wait_write(s, b):
            pltpu.make_async_copy(
                rows_v.at[s, b], out_hbm.at[pl.ds(base, CH)],
                wsems.at[s, b]).wait()

        def do_block(k, s, reuse):
            if reuse:
                for b in range(NB):
                    wait_write(s, b)
            copies = [fire(k * NB + b, s, b) for b in range(NB)]
            for b in range(NB):
                copies[b].wait()
                write(k * NB + b, s, b)

        do_block(0, 0, False)
        do_block(1, 1, False)

        def body(i, carry):
            k = 2 * i + 2
            do_block(k, 0, True)
            do_block(k + 1, 1, True)
            return carry

        lax.fori_loop(0, (NBLK - 2) // 2, body, 0)

        for s in range(2):
            for b in range(NB):
                wait_write(s, b)

    return gather_k


def _ln_body(x_ref, pos_ref, g_ref, b_ref, o_ref):
    x = x_ref[...] + pos_ref[...][None, :, :]

    def norm(v):
        mean = jnp.mean(v, axis=-1, keepdims=True)
        var = jnp.mean(jnp.square(v - mean), axis=-1, keepdims=True)
        return (v - mean) * lax.rsqrt(var + 1e-5)

    y = jnp.concatenate([norm(x[..., :HIDDEN]), norm(x[..., HIDDEN:])], axis=-1)
    o_ref[...] = y * g_ref[...][None, :, :] + b_ref[...][None, :, :]


def _tc_ln(x3, pos, gamma, beta):
    return pl.pallas_call(
        _ln_body,
        grid=(B // BB,),
        in_specs=[
            pl.BlockSpec((BB, L // 2, 128), lambda i: (i, 0, 0)),
            pl.BlockSpec((L // 2, 128), lambda i: (0, 0)),
            pl.BlockSpec((1, 128), lambda i: (0, 0)),
            pl.BlockSpec((1, 128), lambda i: (0, 0)),
        ],
        out_specs=pl.BlockSpec((BB, L // 2, 128), lambda i: (i, 0, 0)),
        out_shape=jax.ShapeDtypeStruct((B, L // 2, 128), jnp.float32),
    )(x3, pos, gamma, beta)


def kernel(input_ids, word_table, pos_table, ln_gamma, ln_beta):
    ids = input_ids.astype(jnp.int32).reshape(NW, NCH, CH)
    gathered = _make_sc_gather()(ids, word_table)
    x3 = gathered.reshape(B, L // 2, 128)
    pos2 = pos_table[:L].reshape(L // 2, 128)
    g2 = jnp.concatenate([ln_gamma, ln_gamma]).reshape(1, 128)
    b2 = jnp.concatenate([ln_beta, ln_beta]).reshape(1, 128)
    y = _tc_ln(x3, pos2, g2, b2)
    return y.reshape(B, L, HIDDEN)
